# baseline (speedup 1.0000x reference)
import functools

import jax
import jax.numpy as jnp
from jax import lax
from jax.experimental import pallas as pl
from jax.experimental.pallas import tpu as pltpu, tpu_sc as plsc

N = 10000
D_IN = 128
D = 256
H = 4
ROWB = 1000
GRID = N // ROWB

NPAD = 10240
DUMMY = 10008
E_PAD = 170496
E4 = E_PAD * 4
EPT_A = E_PAD // 32
EPT_B = E_PAD // 16
CH = 592
NSUP = EPT_B // CH
RK = 16
NRK = CH // RK
NQ = NPAD * 4
DEN_SL = NQ // 16
ACC_SL = NPAD // 16


def _enc_body(nf_ref, w_ref, b_ref, o_ref):
    o_ref[...] = jax.nn.relu(
        jnp.dot(nf_ref[...], w_ref[...], preferred_element_type=jnp.float32)
        + b_ref[...])


def _encoder(nf, w, b):
    return pl.pallas_call(
        _enc_body,
        grid=(GRID,),
        in_specs=[
            pl.BlockSpec((ROWB, D_IN), lambda i: (i, 0)),
            pl.BlockSpec((D_IN, D), lambda i: (0, 0)),
            pl.BlockSpec((1, D), lambda i: (0, 0)),
        ],
        out_specs=pl.BlockSpec((ROWB, D), lambda i: (i, 0)),
        out_shape=jax.ShapeDtypeStruct((N, D), jnp.float32),
    )(nf, w, b.reshape(1, D))


def _pre_body(x_ref, w_ref, a_ref, xl_ref, asd_ref):
    xl = jnp.dot(x_ref[...], w_ref[...], preferred_element_type=jnp.float32)
    xl_ref[...] = xl
    asd_ref[...] = jnp.dot(xl, a_ref[...], preferred_element_type=jnp.float32)


def _layer_pre(x, w, acat):
    return pl.pallas_call(
        _pre_body,
        grid=(GRID,),
        in_specs=[
            pl.BlockSpec((ROWB, D), lambda i: (i, 0)),
            pl.BlockSpec((D, H * D), lambda i: (0, 0)),
            pl.BlockSpec((H * D, 8), lambda i: (0, 0)),
        ],
        out_specs=[
            pl.BlockSpec((ROWB, H * D), lambda i: (i, 0)),
            pl.BlockSpec((ROWB, 8), lambda i: (i, 0)),
        ],
        out_shape=[
            jax.ShapeDtypeStruct((N, H * D), jnp.float32),
            jax.ShapeDtypeStruct((N, 8), jnp.float32),
        ],
    )(x, w, acat)


def _post_body(msg_ref, bias_ref, g_ref, b_ref, x_ref, o_ref):
    t = msg_ref[...] * 0.25 + bias_ref[...]
    m = jnp.mean(t, axis=-1, keepdims=True)
    v = jnp.mean((t - m) ** 2, axis=-1, keepdims=True)
    t = (t - m) * jax.lax.rsqrt(v + 1e-5) * g_ref[...] + b_ref[...]
    o_ref[...] = jax.nn.relu(x_ref[...] + t)


def _layer_post(msg, bias, g, b, x):
    return pl.pallas_call(
        _post_body,
        grid=(GRID,),
        in_specs=[
            pl.BlockSpec((ROWB, D), lambda i: (i, 0)),
            pl.BlockSpec((1, D), lambda i: (0, 0)),
            pl.BlockSpec((1, D), lambda i: (0, 0)),
            pl.BlockSpec((1, D), lambda i: (0, 0)),
            pl.BlockSpec((ROWB, D), lambda i: (i, 0)),
        ],
        out_specs=pl.BlockSpec((ROWB, D), lambda i: (i, 0)),
        out_shape=jax.ShapeDtypeStruct((N, D), jnp.float32),
    )(msg, bias.reshape(1, D), g.reshape(1, D), b.reshape(1, D), x)


def _heads_body(e_ref, w1_ref, b1_ref, w2_ref, b2_ref, o_ref):
    h = jax.nn.relu(
        jnp.dot(e_ref[...], w1_ref[...], preferred_element_type=jnp.float32)
        + b1_ref[...])
    o = jnp.dot(h, w2_ref[...], preferred_element_type=jnp.float32) + b2_ref[...]
    col = jax.lax.broadcasted_iota(jnp.int32, o.shape, 1)
    o_ref[...] = jnp.where((col == 0) | (col == 3), jax.nn.sigmoid(o), o)


def _heads(emb, w1cat, b1cat, w2blk, b2cat):
    return pl.pallas_call(
        _heads_body,
        grid=(GRID,),
        in_specs=[
            pl.BlockSpec((ROWB, D), lambda i: (i, 0)),
            pl.BlockSpec((D, 2 * D), lambda i: (0, 0)),
            pl.BlockSpec((1, 2 * D), lambda i: (0, 0)),
            pl.BlockSpec((2 * D, 8), lambda i: (0, 0)),
            pl.BlockSpec((1, 8), lambda i: (0, 0)),
        ],
        out_specs=pl.BlockSpec((ROWB, 8), lambda i: (i, 0)),
        out_shape=jax.ShapeDtypeStruct((N, 8), jnp.float32),
    )(emb, w1cat, b1cat.reshape(1, 2 * D), w2blk, b2cat)


_SC_MESH = plsc.VectorSubcoreMesh(core_axis_name="c", subcore_axis_name="s")


def _skA_body(asf, adf, src4, dstq, zflat, ex_out, den2,
              den_sh, s4t, dqt, asrc, adst, sem0, sem1):
    c = lax.axis_index("c")
    s = lax.axis_index("s")
    g = c * 16 + s
    e4 = EPT_A * 4
    base = g * e4
    pltpu.sync_copy(src4.at[pl.ds(base, e4)], s4t)
    pltpu.sync_copy(dstq.at[pl.ds(base, e4)], dqt)
    pltpu.async_copy(asf.at[s4t], asrc, sem0).wait()
    pltpu.async_copy(adf.at[dqt], adst, sem1).wait()
    pltpu.sync_copy(zflat, den_sh.at[pl.ds(s * DEN_SL, DEN_SL)])
    plsc.subcore_barrier()

    def body(i, _):
        v = asrc[pl.ds(i * 16, 16)] + adst[pl.ds(i * 16, 16)]
        v = jnp.where(v > 0, v, v * 0.2)
        asrc[pl.ds(i * 16, 16)] = jnp.exp(v)
        return 0
    lax.fori_loop(0, e4 // 16, body, 0)

    pltpu.sync_copy(asrc, den_sh.at[dqt], add=True)
    plsc.subcore_barrier()
    pltpu.sync_copy(asrc, ex_out.at[pl.ds(base, e4)])
    pltpu.sync_copy(den_sh.at[pl.ds(s * DEN_SL, DEN_SL)],
                    den2.at[pl.ds(c * NQ + s * DEN_SL, DEN_SL)])


_sc_edge_softmax = functools.partial(
    pl.kernel,
    out_type=[
        jax.ShapeDtypeStruct((E4,), jnp.float32),
        jax.ShapeDtypeStruct((2 * NQ,), jnp.float32),
    ],
    mesh=_SC_MESH,
    scratch_types=[
        pltpu.VMEM_SHARED((NQ,), jnp.float32),
        pltpu.VMEM((EPT_A * 4,), jnp.int32),
        pltpu.VMEM((EPT_A * 4,), jnp.int32),
        pltpu.VMEM((EPT_A * 4,), jnp.float32),
        pltpu.VMEM((EPT_A * 4,), jnp.float32),
        pltpu.SemaphoreType.DMA,
        pltpu.SemaphoreType.DMA,
    ],
)(_skA_body)


def _skB_body(xlr, exh, rdenh, dstq, dstp, idx8, zrows, out2,
              acc_sh, rd_e, ex_ch, dq_ch, ix_ch, dst_ch,
              rb0, rb1, out0, out1, di0, di1, semg, semr, sems0, sems1):
    c = lax.axis_index("c")
    s = lax.axis_index("s")
    RK4 = RK * 4
    pltpu.sync_copy(zrows, acc_sh.at[pl.ds(s * ACC_SL, ACC_SL)])
    plsc.subcore_barrier()

    def gstart(q, buf):
        pltpu.async_copy(xlr.at[ix_ch.at[pl.ds(q * RK4, RK4)]], buf, semg)

    def gwait(buf):
        pltpu.make_async_copy(xlr.at[ix_ch.at[pl.ds(0, RK4)]], buf, semg).wait()

    def swait(outb, dib, semx):
        pltpu.make_async_copy(outb, acc_sh.at[dib], semx).wait()

    def do_chunk(q, buf, outb, dib, semx):
        @pl.when(q >= 2)
        def _():
            swait(outb, dib, semx)

        def edge_body(e, _):
            q4 = (q * RK + e) * 4
            row = e * 4
            av = ex_ch[pl.ds(q4, 16)]
            for k in range(8):
                acc = jnp.zeros((16,), jnp.float32)
                for h in range(4):
                    acc = acc + av[h] * buf[row + h, pl.ds(k * 16, 16)]
                outb[e, pl.ds(k * 16, 16)] = acc
            return 0
        lax.fori_loop(0, RK, edge_body, 0)
        for i in range(RK // 16):
            dib[pl.ds(i * 16, 16)] = dst_ch[pl.ds(q * RK + i * 16, 16)]
        pltpu.async_copy(outb, acc_sh.at[dib], semx, add=True)

    def sup_body(sup, _):
        base_e = s * EPT_B + sup * CH
        base4 = base_e * 4
        cp_ex = pltpu.async_copy(
            exh.at[pl.ds(base4, CH * 4)], ex_ch.at[pl.ds(0, CH * 4)], semr)
        cp_dq = pltpu.async_copy(dstq.at[pl.ds(base4, CH * 4)], dq_ch, sems0)
        cp_ix = pltpu.async_copy(
            idx8.at[pl.ds(c * E4 + base4, CH * 4)], ix_ch, sems1)
        cp_ds = pltpu.async_copy(dstp.at[pl.ds(base_e, CH)], dst_ch, semg)
        cp_ix.wait()
        cp_ds.wait()
        gstart(0, rb0)
        cp_dq.wait()
        cp_rd = pltpu.async_copy(rdenh.at[dq_ch], rd_e, sems0)
        cp_ex.wait()
        cp_rd.wait()

        def attn_body(i, _):
            ex_ch[pl.ds(i * 16, 16)] = (
                ex_ch[pl.ds(i * 16, 16)] * rd_e[pl.ds(i * 16, 16)])
            return 0
        lax.fori_loop(0, CH * 4 // 16, attn_body, 0)

        def pair_body(rr, _):
            q0 = rr * 2
            q1 = q0 + 1
            gwait(rb0)

            @pl.when(q1 < NRK)
            def _():
                gstart(q1, rb1)
            do_chunk(q0, rb0, out0, di0, sems0)

            @pl.when(q1 < NRK)
            def _():
                gwait(rb1)

                @pl.when(q1 + 1 < NRK)
                def _():
                    gstart(q1 + 1, rb0)
                do_chunk(q1, rb1, out1, di1, sems1)
            return 0
        lax.fori_loop(0, (NRK + 1) // 2, pair_body, 0)
        swait(out0, di0, sems0)
        swait(out1, di1, sems1)
        return 0
    lax.fori_loop(0, NSUP, sup_body, 0)

    plsc.subcore_barrier()
    pltpu.sync_copy(acc_sh.at[pl.ds(s * ACC_SL, ACC_SL)],
                    out2.at[c, pl.ds(s * ACC_SL, ACC_SL)])


_sc_edge_aggregate = functools.partial(
    pl.kernel,
    out_type=jax.ShapeDtypeStruct((2, NPAD, 128), jnp.float32),
    mesh=_SC_MESH,
    scratch_types=[
        pltpu.VMEM_SHARED((NPAD, 128), jnp.float32),
        pltpu.VMEM((CH * 4,), jnp.float32),
        pltpu.VMEM((CH * 4 + 16,), jnp.float32),
        pltpu.VMEM((CH * 4,), jnp.int32),
        pltpu.VMEM((CH * 4,), jnp.int32),
        pltpu.VMEM((CH,), jnp.int32),
        pltpu.VMEM((RK * 4, 128), jnp.float32),
        pltpu.VMEM((RK * 4, 128), jnp.float32),
        pltpu.VMEM((RK, 128), jnp.float32),
        pltpu.VMEM((RK, 128), jnp.float32),
        pltpu.VMEM((RK,), jnp.int32),
        pltpu.VMEM((RK,), jnp.int32),
        pltpu.SemaphoreType.DMA,
        pltpu.SemaphoreType.DMA,
        pltpu.SemaphoreType.DMA,
        pltpu.SemaphoreType.DMA,
    ],
)(_skB_body)


def _rden_body(d_ref, o_ref):
    o_ref[...] = 1.0 / (d_ref[0] + d_ref[1] + 1e-16)


def _rden_finalize(den2):
    return pl.pallas_call(
        _rden_body,
        grid=(1,),
        in_specs=[pl.BlockSpec((2, NQ // 128, 128), lambda i: (0, 0, 0))],
        out_specs=pl.BlockSpec((NQ // 128, 128), lambda i: (0, 0)),
        out_shape=jax.ShapeDtypeStruct((NQ // 128, 128), jnp.float32),
    )(den2.reshape(2, NQ // 128, 128))


def _edge_stage(xl, asd, aux):
    a_s = jnp.pad(asd[:, 0:4], ((0, NPAD - N), (0, 0))).reshape(-1)
    a_d = jnp.pad(asd[:, 4:8], ((0, NPAD - N), (0, 0))).reshape(-1)
    ex, den2 = _sc_edge_softmax(a_s, a_d, aux['src4'], aux['dstq'],
                                aux['zflat'])
    rden = _rden_finalize(den2.reshape(2, NQ)).reshape(-1)
    xlr = xl.reshape(N * 8, 128)
    out2 = _sc_edge_aggregate(xlr, ex, rden, aux['dstq'], aux['dstp'],
                              aux['idx8'], aux['zrows'])
    return jnp.concatenate([out2[0, :N], out2[1, :N]], axis=1)


def kernel(node_features, edge_index, params):
    loop = jnp.arange(N, dtype=jnp.int32)
    e = edge_index.shape[1]
    npad_e = E_PAD - e - N
    srcp = jnp.concatenate([edge_index[0].astype(jnp.int32), loop,
                            jnp.zeros((npad_e,), jnp.int32)])
    dstp = jnp.concatenate([edge_index[1].astype(jnp.int32), loop,
                            jnp.full((npad_e,), DUMMY, jnp.int32)])
    r4 = jnp.arange(4, dtype=jnp.int32)
    aux = {
        'src4': (srcp[:, None] * 4 + r4[None, :]).reshape(-1),
        'dstq': (dstp[:, None] * 4 + r4[None, :]).reshape(-1),
        'idx8': jnp.concatenate([
            (srcp[:, None] * 8 + r4[None, :] * 2 + c).reshape(-1)
            for c in (0, 1)]),
        'dstp': dstp,
        'zflat': jnp.zeros((DEN_SL,), jnp.float32),
        'zrows': jnp.zeros((ACC_SL, 128), jnp.float32),
    }

    x = _encoder(node_features, params['enc_W'], params['enc_b'])

    for lp in params['layers']:
        eye = jnp.eye(H, dtype=jnp.float32)
        acat = jnp.concatenate([
            (eye[:, None, :] * lp['att_src'].T[None, :, :]).reshape(H * D, H),
            (eye[:, None, :] * lp['att_dst'].T[None, :, :]).reshape(H * D, H),
        ], axis=1)
        xl, asd = _layer_pre(x, lp['W'], acat)
        msg = _edge_stage(xl, asd, aux)
        x = _layer_post(msg, lp['bias'], lp['ln_g'], lp['ln_b'], x)

    emb = x
    hp = params['heads']
    w1cat = jnp.concatenate([hp[k]['W1'] for k in
                             ['attrition', 'engagement', 'collaboration', 'anomaly']], axis=1)
    b1cat = jnp.concatenate([hp[k]['b1'] for k in
                             ['attrition', 'engagement', 'collaboration', 'anomaly']])
    w2blk = jnp.zeros((2 * D, 8), jnp.float32)
    for i, k in enumerate(['attrition', 'engagement', 'collaboration', 'anomaly']):
        w2blk = w2blk.at[i * (D // 2):(i + 1) * (D // 2), i].set(hp[k]['W2'][:, 0])
    b2cat = jnp.concatenate(
        [hp[k]['b2'] for k in ['attrition', 'engagement', 'collaboration', 'anomaly']]
        + [jnp.zeros((4,), jnp.float32)]).reshape(1, 8)

    hv = _heads(emb, w1cat, b1cat, w2blk, b2cat)
    return (hv[:, 0:1], hv[:, 1:2], hv[:, 2:3], hv[:, 3:4], emb)


# speedup vs baseline: 1.2014x; 1.2014x over previous
import functools

import jax
import jax.numpy as jnp
from jax import lax
from jax.experimental import pallas as pl
from jax.experimental.pallas import tpu as pltpu, tpu_sc as plsc

N = 10000
D_IN = 128
D = 256
H = 4
ROWB = 1000
GRID = N // ROWB

NPAD = 10240
DUMMY = 10008
E_PAD = 170496
E4 = E_PAD * 4
EPT_A = E_PAD // 32
EPT_B = E_PAD // 16
CH = 592
NSUP = EPT_B // CH
RK = 16
NRK = CH // RK
NQ = NPAD * 4
DEN_SL = NQ // 16
ACC_SL = NPAD // 16


def _enc_body(nf_ref, w_ref, b_ref, o_ref):
    o_ref[...] = jax.nn.relu(
        jnp.dot(nf_ref[...], w_ref[...], preferred_element_type=jnp.float32)
        + b_ref[...])


def _encoder(nf, w, b):
    return pl.pallas_call(
        _enc_body,
        grid=(GRID,),
        in_specs=[
            pl.BlockSpec((ROWB, D_IN), lambda i: (i, 0)),
            pl.BlockSpec((D_IN, D), lambda i: (0, 0)),
            pl.BlockSpec((1, D), lambda i: (0, 0)),
        ],
        out_specs=pl.BlockSpec((ROWB, D), lambda i: (i, 0)),
        out_shape=jax.ShapeDtypeStruct((N, D), jnp.float32),
    )(nf, w, b.reshape(1, D))


def _pre_body(x_ref, w_ref, a_ref, xl_ref, asd_ref):
    xl = jnp.dot(x_ref[...], w_ref[...], preferred_element_type=jnp.float32)
    xl_ref[...] = xl
    asd_ref[...] = jnp.dot(xl, a_ref[...], preferred_element_type=jnp.float32)


def _layer_pre(x, w, acat):
    return pl.pallas_call(
        _pre_body,
        grid=(GRID,),
        in_specs=[
            pl.BlockSpec((ROWB, D), lambda i: (i, 0)),
            pl.BlockSpec((D, H * D), lambda i: (0, 0)),
            pl.BlockSpec((H * D, 8), lambda i: (0, 0)),
        ],
        out_specs=[
            pl.BlockSpec((ROWB, H * D), lambda i: (i, 0)),
            pl.BlockSpec((ROWB, 8), lambda i: (i, 0)),
        ],
        out_shape=[
            jax.ShapeDtypeStruct((N, H * D), jnp.float32),
            jax.ShapeDtypeStruct((N, 8), jnp.float32),
        ],
    )(x, w, acat)


def _post_body(msg_ref, bias_ref, g_ref, b_ref, x_ref, o_ref):
    t = msg_ref[...] * 0.25 + bias_ref[...]
    m = jnp.mean(t, axis=-1, keepdims=True)
    v = jnp.mean((t - m) ** 2, axis=-1, keepdims=True)
    t = (t - m) * jax.lax.rsqrt(v + 1e-5) * g_ref[...] + b_ref[...]
    o_ref[...] = jax.nn.relu(x_ref[...] + t)


def _layer_post(msg, bias, g, b, x):
    return pl.pallas_call(
        _post_body,
        grid=(GRID,),
        in_specs=[
            pl.BlockSpec((ROWB, D), lambda i: (i, 0)),
            pl.BlockSpec((1, D), lambda i: (0, 0)),
            pl.BlockSpec((1, D), lambda i: (0, 0)),
            pl.BlockSpec((1, D), lambda i: (0, 0)),
            pl.BlockSpec((ROWB, D), lambda i: (i, 0)),
        ],
        out_specs=pl.BlockSpec((ROWB, D), lambda i: (i, 0)),
        out_shape=jax.ShapeDtypeStruct((N, D), jnp.float32),
    )(msg, bias.reshape(1, D), g.reshape(1, D), b.reshape(1, D), x)


def _heads_body(e_ref, w1_ref, b1_ref, w2_ref, b2_ref, o_ref):
    h = jax.nn.relu(
        jnp.dot(e_ref[...], w1_ref[...], preferred_element_type=jnp.float32)
        + b1_ref[...])
    o = jnp.dot(h, w2_ref[...], preferred_element_type=jnp.float32) + b2_ref[...]
    col = jax.lax.broadcasted_iota(jnp.int32, o.shape, 1)
    o_ref[...] = jnp.where((col == 0) | (col == 3), jax.nn.sigmoid(o), o)


def _heads(emb, w1cat, b1cat, w2blk, b2cat):
    return pl.pallas_call(
        _heads_body,
        grid=(GRID,),
        in_specs=[
            pl.BlockSpec((ROWB, D), lambda i: (i, 0)),
            pl.BlockSpec((D, 2 * D), lambda i: (0, 0)),
            pl.BlockSpec((1, 2 * D), lambda i: (0, 0)),
            pl.BlockSpec((2 * D, 8), lambda i: (0, 0)),
            pl.BlockSpec((1, 8), lambda i: (0, 0)),
        ],
        out_specs=pl.BlockSpec((ROWB, 8), lambda i: (i, 0)),
        out_shape=jax.ShapeDtypeStruct((N, 8), jnp.float32),
    )(emb, w1cat, b1cat.reshape(1, 2 * D), w2blk, b2cat)


_SC_MESH = plsc.VectorSubcoreMesh(core_axis_name="c", subcore_axis_name="s")


def _skA_body(asf, adf, src4, dstq, zflat, ex_out, den2,
              den_sh, s4t, dqt, asrc, adst, sem0, sem1):
    c = lax.axis_index("c")
    s = lax.axis_index("s")
    g = c * 16 + s
    e4 = EPT_A * 4
    base = g * e4
    pltpu.sync_copy(src4.at[pl.ds(base, e4)], s4t)
    pltpu.sync_copy(dstq.at[pl.ds(base, e4)], dqt)
    pltpu.async_copy(asf.at[s4t], asrc, sem0).wait()
    pltpu.async_copy(adf.at[dqt], adst, sem1).wait()
    pltpu.sync_copy(zflat, den_sh.at[pl.ds(s * DEN_SL, DEN_SL)])
    plsc.subcore_barrier()

    @plsc.parallel_loop(0, e4 // 16, 1, unroll=4)
    def body(i):
        v = asrc[pl.ds(i * 16, 16)] + adst[pl.ds(i * 16, 16)]
        v = jnp.where(v > 0, v, v * 0.2)
        asrc[pl.ds(i * 16, 16)] = jnp.exp(v)

    pltpu.sync_copy(asrc, den_sh.at[dqt], add=True)
    plsc.subcore_barrier()
    pltpu.sync_copy(asrc, ex_out.at[pl.ds(base, e4)])
    pltpu.sync_copy(den_sh.at[pl.ds(s * DEN_SL, DEN_SL)],
                    den2.at[pl.ds(c * NQ + s * DEN_SL, DEN_SL)])


_sc_edge_softmax = functools.partial(
    pl.kernel,
    out_type=[
        jax.ShapeDtypeStruct((E4,), jnp.float32),
        jax.ShapeDtypeStruct((2 * NQ,), jnp.float32),
    ],
    mesh=_SC_MESH,
    scratch_types=[
        pltpu.VMEM_SHARED((NQ,), jnp.float32),
        pltpu.VMEM((EPT_A * 4,), jnp.int32),
        pltpu.VMEM((EPT_A * 4,), jnp.int32),
        pltpu.VMEM((EPT_A * 4,), jnp.float32),
        pltpu.VMEM((EPT_A * 4,), jnp.float32),
        pltpu.SemaphoreType.DMA,
        pltpu.SemaphoreType.DMA,
    ],
)(_skA_body)


def _skB_body(xlr, exh, rdenh, dstq, dstp, idx8, zrows, out2,
              acc_sh, rd_e, ex_ch, dq_ch, ix_ch, dst_ch,
              rb0, rb1, out0, out1, di0, di1, semg, semr, sems0, sems1):
    c = lax.axis_index("c")
    s = lax.axis_index("s")
    RK4 = RK * 4
    pltpu.sync_copy(zrows, acc_sh.at[pl.ds(s * ACC_SL, ACC_SL)])
    plsc.subcore_barrier()

    def gstart(q, buf):
        pltpu.async_copy(xlr.at[ix_ch.at[pl.ds(q * RK4, RK4)]], buf, semg)

    def gwait(buf):
        pltpu.make_async_copy(xlr.at[ix_ch.at[pl.ds(0, RK4)]], buf, semg).wait()

    def swait(outb, dib, semx):
        pltpu.make_async_copy(outb, acc_sh.at[dib], semx).wait()

    def do_chunk(q, buf, outb, dib, semx):
        @pl.when(q >= 2)
        def _():
            swait(outb, dib, semx)

        @plsc.parallel_loop(0, RK, 1, unroll=4)
        def edge_body(e):
            q4 = (q * RK + e) * 4
            row = e * 4
            av = ex_ch[pl.ds(q4, 16)]
            for k in range(8):
                acc = jnp.zeros((16,), jnp.float32)
                for h in range(4):
                    acc = acc + av[h] * buf[row + h, pl.ds(k * 16, 16)]
                outb[e, pl.ds(k * 16, 16)] = acc
        for i in range(RK // 16):
            dib[pl.ds(i * 16, 16)] = dst_ch[pl.ds(q * RK + i * 16, 16)]
        pltpu.async_copy(outb, acc_sh.at[dib], semx, add=True)

    def sup_body(sup, _):
        base_e = s * EPT_B + sup * CH
        base4 = base_e * 4
        cp_ex = pltpu.async_copy(
            exh.at[pl.ds(base4, CH * 4)], ex_ch.at[pl.ds(0, CH * 4)], semr)
        cp_dq = pltpu.async_copy(dstq.at[pl.ds(base4, CH * 4)], dq_ch, sems0)
        cp_ix = pltpu.async_copy(
            idx8.at[pl.ds(c * E4 + base4, CH * 4)], ix_ch, sems1)
        cp_ds = pltpu.async_copy(dstp.at[pl.ds(base_e, CH)], dst_ch, semg)
        cp_ix.wait()
        cp_ds.wait()
        gstart(0, rb0)
        cp_dq.wait()
        cp_rd = pltpu.async_copy(rdenh.at[dq_ch], rd_e, sems0)
        cp_ex.wait()
        cp_rd.wait()

        @plsc.parallel_loop(0, CH * 4 // 16, 1, unroll=4)
        def attn_body(i):
            ex_ch[pl.ds(i * 16, 16)] = (
                ex_ch[pl.ds(i * 16, 16)] * rd_e[pl.ds(i * 16, 16)])

        def pair_body(rr, _):
            q0 = rr * 2
            q1 = q0 + 1
            gwait(rb0)

            @pl.when(q1 < NRK)
            def _():
                gstart(q1, rb1)
            do_chunk(q0, rb0, out0, di0, sems0)

            @pl.when(q1 < NRK)
            def _():
                gwait(rb1)

                @pl.when(q1 + 1 < NRK)
                def _():
                    gstart(q1 + 1, rb0)
                do_chunk(q1, rb1, out1, di1, sems1)
            return 0
        lax.fori_loop(0, (NRK + 1) // 2, pair_body, 0)
        swait(out0, di0, sems0)
        swait(out1, di1, sems1)
        return 0
    lax.fori_loop(0, NSUP, sup_body, 0)

    plsc.subcore_barrier()
    pltpu.sync_copy(acc_sh.at[pl.ds(s * ACC_SL, ACC_SL)],
                    out2.at[c, pl.ds(s * ACC_SL, ACC_SL)])


_sc_edge_aggregate = functools.partial(
    pl.kernel,
    out_type=jax.ShapeDtypeStruct((2, NPAD, 128), jnp.float32),
    mesh=_SC_MESH,
    scratch_types=[
        pltpu.VMEM_SHARED((NPAD, 128), jnp.float32),
        pltpu.VMEM((CH * 4,), jnp.float32),
        pltpu.VMEM((CH * 4 + 16,), jnp.float32),
        pltpu.VMEM((CH * 4,), jnp.int32),
        pltpu.VMEM((CH * 4,), jnp.int32),
        pltpu.VMEM((CH,), jnp.int32),
        pltpu.VMEM((RK * 4, 128), jnp.float32),
        pltpu.VMEM((RK * 4, 128), jnp.float32),
        pltpu.VMEM((RK, 128), jnp.float32),
        pltpu.VMEM((RK, 128), jnp.float32),
        pltpu.VMEM((RK,), jnp.int32),
        pltpu.VMEM((RK,), jnp.int32),
        pltpu.SemaphoreType.DMA,
        pltpu.SemaphoreType.DMA,
        pltpu.SemaphoreType.DMA,
        pltpu.SemaphoreType.DMA,
    ],
)(_skB_body)


def _rden_body(d_ref, o_ref):
    o_ref[...] = 1.0 / (d_ref[0] + d_ref[1] + 1e-16)


def _rden_finalize(den2):
    return pl.pallas_call(
        _rden_body,
        grid=(1,),
        in_specs=[pl.BlockSpec((2, NQ // 128, 128), lambda i: (0, 0, 0))],
        out_specs=pl.BlockSpec((NQ // 128, 128), lambda i: (0, 0)),
        out_shape=jax.ShapeDtypeStruct((NQ // 128, 128), jnp.float32),
    )(den2.reshape(2, NQ // 128, 128))


def _edge_stage(xl, asd, aux):
    a_s = jnp.pad(asd[:, 0:4], ((0, NPAD - N), (0, 0))).reshape(-1)
    a_d = jnp.pad(asd[:, 4:8], ((0, NPAD - N), (0, 0))).reshape(-1)
    ex, den2 = _sc_edge_softmax(a_s, a_d, aux['src4'], aux['dstq'],
                                aux['zflat'])
    rden = _rden_finalize(den2.reshape(2, NQ)).reshape(-1)
    xlr = xl.reshape(N * 8, 128)
    out2 = _sc_edge_aggregate(xlr, ex, rden, aux['dstq'], aux['dstp'],
                              aux['idx8'], aux['zrows'])
    return jnp.concatenate([out2[0, :N], out2[1, :N]], axis=1)


def kernel(node_features, edge_index, params):
    loop = jnp.arange(N, dtype=jnp.int32)
    e = edge_index.shape[1]
    npad_e = E_PAD - e - N
    srcp = jnp.concatenate([edge_index[0].astype(jnp.int32), loop,
                            jnp.zeros((npad_e,), jnp.int32)])
    dstp = jnp.concatenate([edge_index[1].astype(jnp.int32), loop,
                            jnp.full((npad_e,), DUMMY, jnp.int32)])
    r4 = jnp.arange(4, dtype=jnp.int32)
    aux = {
        'src4': (srcp[:, None] * 4 + r4[None, :]).reshape(-1),
        'dstq': (dstp[:, None] * 4 + r4[None, :]).reshape(-1),
        'idx8': jnp.concatenate([
            (srcp[:, None] * 8 + r4[None, :] * 2 + c).reshape(-1)
            for c in (0, 1)]),
        'dstp': dstp,
        'zflat': jnp.zeros((DEN_SL,), jnp.float32),
        'zrows': jnp.zeros((ACC_SL, 128), jnp.float32),
    }

    x = _encoder(node_features, params['enc_W'], params['enc_b'])

    for lp in params['layers']:
        eye = jnp.eye(H, dtype=jnp.float32)
        acat = jnp.concatenate([
            (eye[:, None, :] * lp['att_src'].T[None, :, :]).reshape(H * D, H),
            (eye[:, None, :] * lp['att_dst'].T[None, :, :]).reshape(H * D, H),
        ], axis=1)
        xl, asd = _layer_pre(x, lp['W'], acat)
        msg = _edge_stage(xl, asd, aux)
        x = _layer_post(msg, lp['bias'], lp['ln_g'], lp['ln_b'], x)

    emb = x
    hp = params['heads']
    w1cat = jnp.concatenate([hp[k]['W1'] for k in
                             ['attrition', 'engagement', 'collaboration', 'anomaly']], axis=1)
    b1cat = jnp.concatenate([hp[k]['b1'] for k in
                             ['attrition', 'engagement', 'collaboration', 'anomaly']])
    w2blk = jnp.zeros((2 * D, 8), jnp.float32)
    for i, k in enumerate(['attrition', 'engagement', 'collaboration', 'anomaly']):
        w2blk = w2blk.at[i * (D // 2):(i + 1) * (D // 2), i].set(hp[k]['W2'][:, 0])
    b2cat = jnp.concatenate(
        [hp[k]['b2'] for k in ['attrition', 'engagement', 'collaboration', 'anomaly']]
        + [jnp.zeros((4,), jnp.float32)]).reshape(1, 8)

    hv = _heads(emb, w1cat, b1cat, w2blk, b2cat)
    return (hv[:, 0:1], hv[:, 1:2], hv[:, 2:3], hv[:, 3:4], emb)


# speedup vs baseline: 1.2375x; 1.0301x over previous
import functools

import jax
import jax.numpy as jnp
from jax import lax
from jax.experimental import pallas as pl
from jax.experimental.pallas import tpu as pltpu, tpu_sc as plsc

N = 10000
D_IN = 128
D = 256
H = 4
ROWB = 1000
GRID = N // ROWB

NPAD = 10240
DUMMY = 10008
E_PAD = 170496
E4 = E_PAD * 4
EPT_A = E_PAD // 32
EPT_B = E_PAD // 16
CH = 592
NSUP = EPT_B // CH
RK = 16
NRK = CH // RK
NQ = NPAD * 4
DEN_SL = NQ // 16
ACC_SL = NPAD // 16


def _enc_body(nf_ref, w_ref, b_ref, o_ref):
    o_ref[...] = jax.nn.relu(
        jnp.dot(nf_ref[...], w_ref[...], preferred_element_type=jnp.float32)
        + b_ref[...])


def _encoder(nf, w, b):
    return pl.pallas_call(
        _enc_body,
        grid=(GRID,),
        in_specs=[
            pl.BlockSpec((ROWB, D_IN), lambda i: (i, 0)),
            pl.BlockSpec((D_IN, D), lambda i: (0, 0)),
            pl.BlockSpec((1, D), lambda i: (0, 0)),
        ],
        out_specs=pl.BlockSpec((ROWB, D), lambda i: (i, 0)),
        out_shape=jax.ShapeDtypeStruct((N, D), jnp.float32),
    )(nf, w, b.reshape(1, D))


def _pre_body(x_ref, w_ref, a_ref, xl_ref, asd_ref):
    xl = jnp.dot(x_ref[...], w_ref[...], preferred_element_type=jnp.float32)
    xl_ref[...] = xl
    asd_ref[...] = jnp.dot(xl, a_ref[...], preferred_element_type=jnp.float32)


def _layer_pre(x, w, acat):
    return pl.pallas_call(
        _pre_body,
        grid=(GRID,),
        in_specs=[
            pl.BlockSpec((ROWB, D), lambda i: (i, 0)),
            pl.BlockSpec((D, H * D), lambda i: (0, 0)),
            pl.BlockSpec((H * D, 8), lambda i: (0, 0)),
        ],
        out_specs=[
            pl.BlockSpec((ROWB, H * D), lambda i: (i, 0)),
            pl.BlockSpec((ROWB, 8), lambda i: (i, 0)),
        ],
        out_shape=[
            jax.ShapeDtypeStruct((N, H * D), jnp.float32),
            jax.ShapeDtypeStruct((N, 8), jnp.float32),
        ],
    )(x, w, acat)


def _post_body(m0_ref, m1_ref, bias_ref, g_ref, b_ref, x_ref, o_ref):
    t = jnp.concatenate([m0_ref[0], m1_ref[0]], axis=1) * 0.25 + bias_ref[...]
    m = jnp.mean(t, axis=-1, keepdims=True)
    v = jnp.mean((t - m) ** 2, axis=-1, keepdims=True)
    t = (t - m) * jax.lax.rsqrt(v + 1e-5) * g_ref[...] + b_ref[...]
    o_ref[...] = jax.nn.relu(x_ref[...] + t)


def _layer_post(out2, bias, g, b, x):
    return pl.pallas_call(
        _post_body,
        grid=(GRID,),
        in_specs=[
            pl.BlockSpec((1, ROWB, 128), lambda i: (0, i, 0)),
            pl.BlockSpec((1, ROWB, 128), lambda i: (1, i, 0)),
            pl.BlockSpec((1, D), lambda i: (0, 0)),
            pl.BlockSpec((1, D), lambda i: (0, 0)),
            pl.BlockSpec((1, D), lambda i: (0, 0)),
            pl.BlockSpec((ROWB, D), lambda i: (i, 0)),
        ],
        out_specs=pl.BlockSpec((ROWB, D), lambda i: (i, 0)),
        out_shape=jax.ShapeDtypeStruct((N, D), jnp.float32),
    )(out2, out2, bias.reshape(1, D), g.reshape(1, D), b.reshape(1, D), x)


def _heads_body(e_ref, w1_ref, b1_ref, w2_ref, b2_ref, o_ref):
    h = jax.nn.relu(
        jnp.dot(e_ref[...], w1_ref[...], preferred_element_type=jnp.float32)
        + b1_ref[...])
    o = jnp.dot(h, w2_ref[...], preferred_element_type=jnp.float32) + b2_ref[...]
    col = jax.lax.broadcasted_iota(jnp.int32, o.shape, 1)
    o_ref[...] = jnp.where((col == 0) | (col == 3), jax.nn.sigmoid(o), o)


def _heads(emb, w1cat, b1cat, w2blk, b2cat):
    return pl.pallas_call(
        _heads_body,
        grid=(GRID,),
        in_specs=[
            pl.BlockSpec((ROWB, D), lambda i: (i, 0)),
            pl.BlockSpec((D, 2 * D), lambda i: (0, 0)),
            pl.BlockSpec((1, 2 * D), lambda i: (0, 0)),
            pl.BlockSpec((2 * D, 8), lambda i: (0, 0)),
            pl.BlockSpec((1, 8), lambda i: (0, 0)),
        ],
        out_specs=pl.BlockSpec((ROWB, 8), lambda i: (i, 0)),
        out_shape=jax.ShapeDtypeStruct((N, 8), jnp.float32),
    )(emb, w1cat, b1cat.reshape(1, 2 * D), w2blk, b2cat)


_SC_MESH = plsc.VectorSubcoreMesh(core_axis_name="c", subcore_axis_name="s")


def _skA_body(asf, adf, src4, dstq, zflat, ex_out, den2,
              den_sh, s4t, dqt, asrc, adst, sem0, sem1):
    c = lax.axis_index("c")
    s = lax.axis_index("s")
    g = c * 16 + s
    e4 = EPT_A * 4
    base = g * e4
    pltpu.sync_copy(src4.at[pl.ds(base, e4)], s4t)
    pltpu.sync_copy(dstq.at[pl.ds(base, e4)], dqt)
    cp_a = pltpu.async_copy(asf.at[s4t], asrc, sem0)
    cp_b = pltpu.async_copy(adf.at[dqt], adst, sem1)
    cp_a.wait()
    cp_b.wait()
    pltpu.sync_copy(zflat, den_sh.at[pl.ds(s * DEN_SL, DEN_SL)])
    plsc.subcore_barrier()

    @plsc.parallel_loop(0, e4 // 16, 1, unroll=4)
    def body(i):
        v = asrc[pl.ds(i * 16, 16)] + adst[pl.ds(i * 16, 16)]
        v = jnp.where(v > 0, v, v * 0.2)
        asrc[pl.ds(i * 16, 16)] = jnp.exp(v)

    pltpu.sync_copy(asrc, den_sh.at[dqt], add=True)
    plsc.subcore_barrier()
    pltpu.sync_copy(asrc, ex_out.at[pl.ds(base, e4)])
    pltpu.sync_copy(den_sh.at[pl.ds(s * DEN_SL, DEN_SL)],
                    den2.at[pl.ds(c * NQ + s * DEN_SL, DEN_SL)])


_sc_edge_softmax = functools.partial(
    pl.kernel,
    out_type=[
        jax.ShapeDtypeStruct((E4,), jnp.float32),
        jax.ShapeDtypeStruct((2 * NQ,), jnp.float32),
    ],
    mesh=_SC_MESH,
    scratch_types=[
        pltpu.VMEM_SHARED((NQ,), jnp.float32),
        pltpu.VMEM((EPT_A * 4,), jnp.int32),
        pltpu.VMEM((EPT_A * 4,), jnp.int32),
        pltpu.VMEM((EPT_A * 4,), jnp.float32),
        pltpu.VMEM((EPT_A * 4,), jnp.float32),
        pltpu.SemaphoreType.DMA,
        pltpu.SemaphoreType.DMA,
    ],
)(_skA_body)


def _skB_body(xlr, exh, rdenh, dstq, dstp, idx8, zrows, out2,
              acc_sh, rd_e, ex_ch, dq_ch, ix_ch, dst_ch,
              rb0, rb1, out0, out1, di0, di1, semg, semr, sems0, sems1):
    c = lax.axis_index("c")
    s = lax.axis_index("s")
    RK4 = RK * 4
    pltpu.sync_copy(zrows, acc_sh.at[pl.ds(s * ACC_SL, ACC_SL)])
    plsc.subcore_barrier()

    def gstart(q, buf):
        pltpu.async_copy(xlr.at[ix_ch.at[pl.ds(q * RK4, RK4)]], buf, semg)

    def gwait(buf):
        pltpu.make_async_copy(xlr.at[ix_ch.at[pl.ds(0, RK4)]], buf, semg).wait()

    def swait(outb, dib, semx):
        pltpu.make_async_copy(outb, acc_sh.at[dib], semx).wait()

    def do_chunk(q, buf, outb, dib, semx):
        @pl.when(q >= 2)
        def _():
            swait(outb, dib, semx)

        @plsc.parallel_loop(0, RK, 1, unroll=4)
        def edge_body(e):
            q4 = (q * RK + e) * 4
            row = e * 4
            av = ex_ch[pl.ds(q4, 16)]
            for k in range(8):
                acc = jnp.zeros((16,), jnp.float32)
                for h in range(4):
                    acc = acc + av[h] * buf[row + h, pl.ds(k * 16, 16)]
                outb[e, pl.ds(k * 16, 16)] = acc
        for i in range(RK // 16):
            dib[pl.ds(i * 16, 16)] = dst_ch[pl.ds(q * RK + i * 16, 16)]
        pltpu.async_copy(outb, acc_sh.at[dib], semx, add=True)

    def sup_body(sup, _):
        base_e = s * EPT_B + sup * CH
        base4 = base_e * 4
        cp_ex = pltpu.async_copy(
            exh.at[pl.ds(base4, CH * 4)], ex_ch.at[pl.ds(0, CH * 4)], semr)
        cp_dq = pltpu.async_copy(dstq.at[pl.ds(base4, CH * 4)], dq_ch, sems0)
        cp_ix = pltpu.async_copy(
            idx8.at[pl.ds(c * E4 + base4, CH * 4)], ix_ch, sems1)
        cp_ds = pltpu.async_copy(dstp.at[pl.ds(base_e, CH)], dst_ch, semg)
        cp_ix.wait()
        cp_ds.wait()
        gstart(0, rb0)
        cp_dq.wait()
        cp_rd = pltpu.async_copy(rdenh.at[dq_ch], rd_e, sems0)
        cp_ex.wait()
        cp_rd.wait()

        @plsc.parallel_loop(0, CH * 4 // 16, 1, unroll=4)
        def attn_body(i):
            ex_ch[pl.ds(i * 16, 16)] = (
                ex_ch[pl.ds(i * 16, 16)] * rd_e[pl.ds(i * 16, 16)])

        def pair_body(rr, _):
            q0 = rr * 2
            q1 = q0 + 1
            gwait(rb0)

            @pl.when(q1 < NRK)
            def _():
                gstart(q1, rb1)
            do_chunk(q0, rb0, out0, di0, sems0)

            @pl.when(q1 < NRK)
            def _():
                gwait(rb1)

                @pl.when(q1 + 1 < NRK)
                def _():
                    gstart(q1 + 1, rb0)
                do_chunk(q1, rb1, out1, di1, sems1)
            return 0
        lax.fori_loop(0, (NRK + 1) // 2, pair_body, 0)
        swait(out0, di0, sems0)
        swait(out1, di1, sems1)
        return 0
    lax.fori_loop(0, NSUP, sup_body, 0)

    plsc.subcore_barrier()
    pltpu.sync_copy(acc_sh.at[pl.ds(s * ACC_SL, ACC_SL)],
                    out2.at[c, pl.ds(s * ACC_SL, ACC_SL)])


_sc_edge_aggregate = functools.partial(
    pl.kernel,
    out_type=jax.ShapeDtypeStruct((2, NPAD, 128), jnp.float32),
    mesh=_SC_MESH,
    scratch_types=[
        pltpu.VMEM_SHARED((NPAD, 128), jnp.float32),
        pltpu.VMEM((CH * 4,), jnp.float32),
        pltpu.VMEM((CH * 4 + 16,), jnp.float32),
        pltpu.VMEM((CH * 4,), jnp.int32),
        pltpu.VMEM((CH * 4,), jnp.int32),
        pltpu.VMEM((CH,), jnp.int32),
        pltpu.VMEM((RK * 4, 128), jnp.float32),
        pltpu.VMEM((RK * 4, 128), jnp.float32),
        pltpu.VMEM((RK, 128), jnp.float32),
        pltpu.VMEM((RK, 128), jnp.float32),
        pltpu.VMEM((RK,), jnp.int32),
        pltpu.VMEM((RK,), jnp.int32),
        pltpu.SemaphoreType.DMA,
        pltpu.SemaphoreType.DMA,
        pltpu.SemaphoreType.DMA,
        pltpu.SemaphoreType.DMA,
    ],
)(_skB_body)


def _rden_body(d_ref, o_ref):
    o_ref[...] = 1.0 / (d_ref[0] + d_ref[1] + 1e-16)


def _rden_finalize(den2):
    return pl.pallas_call(
        _rden_body,
        grid=(1,),
        in_specs=[pl.BlockSpec((2, NQ // 128, 128), lambda i: (0, 0, 0))],
        out_specs=pl.BlockSpec((NQ // 128, 128), lambda i: (0, 0)),
        out_shape=jax.ShapeDtypeStruct((NQ // 128, 128), jnp.float32),
    )(den2.reshape(2, NQ // 128, 128))


def _edge_stage(xl, asd, aux):
    a_s = jnp.pad(asd[:, 0:4], ((0, NPAD - N), (0, 0))).reshape(-1)
    a_d = jnp.pad(asd[:, 4:8], ((0, NPAD - N), (0, 0))).reshape(-1)
    ex, den2 = _sc_edge_softmax(a_s, a_d, aux['src4'], aux['dstq'],
                                aux['zflat'])
    rden = _rden_finalize(den2.reshape(2, NQ)).reshape(-1)
    xlr = xl.reshape(N * 8, 128)
    return _sc_edge_aggregate(xlr, ex, rden, aux['dstq'], aux['dstp'],
                              aux['idx8'], aux['zrows'])


def kernel(node_features, edge_index, params):
    loop = jnp.arange(N, dtype=jnp.int32)
    e = edge_index.shape[1]
    npad_e = E_PAD - e - N
    srcp = jnp.concatenate([edge_index[0].astype(jnp.int32), loop,
                            jnp.zeros((npad_e,), jnp.int32)])
    dstp = jnp.concatenate([edge_index[1].astype(jnp.int32), loop,
                            jnp.full((npad_e,), DUMMY, jnp.int32)])
    r4 = jnp.arange(4, dtype=jnp.int32)
    aux = {
        'src4': (srcp[:, None] * 4 + r4[None, :]).reshape(-1),
        'dstq': (dstp[:, None] * 4 + r4[None, :]).reshape(-1),
        'idx8': jnp.concatenate([
            (srcp[:, None] * 8 + r4[None, :] * 2 + c).reshape(-1)
            for c in (0, 1)]),
        'dstp': dstp,
        'zflat': jnp.zeros((DEN_SL,), jnp.float32),
        'zrows': jnp.zeros((ACC_SL, 128), jnp.float32),
    }

    x = _encoder(node_features, params['enc_W'], params['enc_b'])

    for lp in params['layers']:
        eye = jnp.eye(H, dtype=jnp.float32)
        acat = jnp.concatenate([
            (eye[:, None, :] * lp['att_src'].T[None, :, :]).reshape(H * D, H),
            (eye[:, None, :] * lp['att_dst'].T[None, :, :]).reshape(H * D, H),
        ], axis=1)
        xl, asd = _layer_pre(x, lp['W'], acat)
        msg = _edge_stage(xl, asd, aux)
        x = _layer_post(msg, lp['bias'], lp['ln_g'], lp['ln_b'], x)

    emb = x
    hp = params['heads']
    w1cat = jnp.concatenate([hp[k]['W1'] for k in
                             ['attrition', 'engagement', 'collaboration', 'anomaly']], axis=1)
    b1cat = jnp.concatenate([hp[k]['b1'] for k in
                             ['attrition', 'engagement', 'collaboration', 'anomaly']])
    w2blk = jnp.zeros((2 * D, 8), jnp.float32)
    for i, k in enumerate(['attrition', 'engagement', 'collaboration', 'anomaly']):
        w2blk = w2blk.at[i * (D // 2):(i + 1) * (D // 2), i].set(hp[k]['W2'][:, 0])
    b2cat = jnp.concatenate(
        [hp[k]['b2'] for k in ['attrition', 'engagement', 'collaboration', 'anomaly']]
        + [jnp.zeros((4,), jnp.float32)]).reshape(1, 8)

    hv = _heads(emb, w1cat, b1cat, w2blk, b2cat)
    return (hv[:, 0:1], hv[:, 1:2], hv[:, 2:3], hv[:, 3:4], emb)
